# baseline (speedup 1.0000x reference)
import functools

import jax
import jax.numpy as jnp
import numpy as np
from jax import lax
from jax.experimental import pallas as pl
from jax.experimental.pallas import tpu as pltpu
from jax.experimental.pallas import tpu_sc as plsc

N_EMBED = 4096
NUM_EXPERTS = 64
TOP_K = 8
TOKENS = 8192

_BLOCK_T = 1024
_NW = 32
_TPW = TOKENS // _NW
_L = 16

_GAUSS_T = np.ascontiguousarray(
    np.asarray(
        jax.device_get(
            jax.random.normal(
                jax.random.key(42), (TOKENS, NUM_EXPERTS), dtype=jnp.float32
            )
        )
    ).T
)


def _noisy_block(x_ref, w_ref, b_ref, g_ref, out_ref):
    acc_t = jax.lax.dot_general(
        w_ref[...],
        x_ref[...],
        dimension_numbers=(((0,), (1,)), ((), ())),
        preferred_element_type=jnp.float32,
    )
    acc_t = acc_t + b_ref[...]
    logits = acc_t[:NUM_EXPERTS, :]
    noise_logits = acc_t[NUM_EXPERTS:, :]
    out_ref[...] = logits + g_ref[...] * jax.nn.softplus(noise_logits)


@jax.jit
def _tc_noisy(x, w_comb, b_comb):
    gauss_t = jnp.asarray(_GAUSS_T)
    n_blocks = TOKENS // _BLOCK_T
    return pl.pallas_call(
        _noisy_block,
        grid=(n_blocks,),
        in_specs=[
            pl.BlockSpec((_BLOCK_T, N_EMBED), lambda i: (i, 0)),
            pl.BlockSpec((N_EMBED, 2 * NUM_EXPERTS), lambda i: (0, 0)),
            pl.BlockSpec((2 * NUM_EXPERTS, 1), lambda i: (0, 0)),
            pl.BlockSpec((NUM_EXPERTS, _BLOCK_T), lambda i: (0, i)),
        ],
        out_specs=pl.BlockSpec((NUM_EXPERTS, _BLOCK_T), lambda i: (0, i)),
        out_shape=jax.ShapeDtypeStruct((NUM_EXPERTS, TOKENS), jnp.float32),
    )(x, w_comb, b_comb, gauss_t)


def _sc_body(noisy_hbm, probs_hbm, idx_hbm, slab, pslab, islab):
    wid = lax.axis_index("s") * 2 + lax.axis_index("c")
    base = wid * _TPW
    pltpu.sync_copy(noisy_hbm.at[:, pl.ds(base, _TPW)], slab)

    zero = jnp.zeros((_L,), jnp.float32)
    neg_inf = jnp.full((_L,), -jnp.inf, jnp.float32)
    lane_iota = lax.iota(jnp.int32, _L)

    def group(g, carry):
        s = pl.ds(g * _L, _L)
        m = slab[0, s]
        for e in range(1, NUM_EXPERTS):
            m = jnp.maximum(m, slab[e, s])
        ims = []
        vals = []
        for j in range(TOP_K):
            im = jnp.full((_L,), NUM_EXPERTS, jnp.int32)
            for e in range(NUM_EXPERTS - 1, -1, -1):
                im = jnp.where(slab[e, s] == m, e, im)
            islab[j, s] = im
            ims.append(im)
            vals.append(m)
            if j < TOP_K - 1:
                nm = neg_inf
                for e in range(NUM_EXPERTS):
                    v = slab[e, s]
                    v = jnp.where(im == e, neg_inf, v)
                    slab[e, s] = v
                    nm = jnp.maximum(nm, v)
                m = nm
        exps = [jnp.exp(v - vals[0]) for v in vals]
        den = exps[0]
        for t in exps[1:]:
            den = den + t
        inv = 1.0 / den
        pvals = [t * inv for t in exps]
        for e in range(NUM_EXPERTS):
            p = zero
            for j in range(TOP_K):
                p = jnp.where(ims[j] == e, pvals[j], p)
            pslab[e, s] = p
        return carry

    lax.fori_loop(0, _TPW // _L, group, 0)

    pltpu.sync_copy(pslab, probs_hbm.at[:, pl.ds(base, _TPW)])
    pltpu.sync_copy(islab, idx_hbm.at[:, pl.ds(base, _TPW)])


@jax.jit
def _sc_route(noisy_t):
    mesh = plsc.VectorSubcoreMesh(core_axis_name="c", subcore_axis_name="s")
    run = functools.partial(
        pl.kernel,
        out_type=(
            jax.ShapeDtypeStruct((NUM_EXPERTS, TOKENS), jnp.float32),
            jax.ShapeDtypeStruct((TOP_K, TOKENS), jnp.int32),
        ),
        mesh=mesh,
        scratch_types=[
            pltpu.VMEM((NUM_EXPERTS, _TPW), jnp.float32),
            pltpu.VMEM((NUM_EXPERTS, _TPW), jnp.float32),
            pltpu.VMEM((TOP_K, _TPW), jnp.int32),
        ],
    )(_sc_body)
    return run(noisy_t)


def kernel(x, W_router, b_router, W_noise, b_noise):
    w_comb = jnp.concatenate([W_router.T, W_noise.T], axis=1)
    b_comb = jnp.concatenate([b_router, b_noise])[:, None]
    noisy_t = _tc_noisy(x, w_comb, b_comb)
    probs_t, idx_t = _sc_route(noisy_t)
    return (probs_t.T, idx_t.T)


# speedup vs baseline: 3.0251x; 3.0251x over previous
import functools

import jax
import jax.numpy as jnp
import numpy as np
from jax.experimental import pallas as pl

N_EMBED = 4096
NUM_EXPERTS = 64
TOP_K = 8
TOKENS = 8192

_BLOCK_T = 1024


def _router_block(x_ref, w_ref, b_ref, g_ref, out_ref, idx_ref):
    acc_t = jax.lax.dot_general(
        w_ref[...],
        x_ref[...],
        dimension_numbers=(((0,), (1,)), ((), ())),
        preferred_element_type=jnp.float32,
        precision=jax.lax.Precision.DEFAULT,
    )
    acc_t = acc_t + b_ref[...]
    logits = acc_t[:NUM_EXPERTS, :]
    noise_logits = acc_t[NUM_EXPERTS:, :]
    noisy = logits + g_ref[...] * jax.nn.softplus(noise_logits)

    iota_e = jax.lax.broadcasted_iota(jnp.int32, noisy.shape, 0)
    work = noisy
    vals = []
    idxs = []
    for _ in range(TOP_K):
        m = jnp.max(work, axis=0, keepdims=True)
        amax = jnp.min(
            jnp.where(work == m, iota_e, NUM_EXPERTS), axis=0, keepdims=True
        )
        work = jnp.where(iota_e == amax, -jnp.inf, work)
        vals.append(m)
        idxs.append(amax)

    exps = [jnp.exp(v - vals[0]) for v in vals]
    denom = exps[0]
    for e in exps[1:]:
        denom = denom + e
    inv = 1.0 / denom

    out = jnp.zeros(noisy.shape, jnp.float32)
    for e, v in zip(exps, idxs):
        out = out + jnp.where(iota_e == v, e * inv, 0.0)
    out_ref[...] = out
    idx_ref[...] = jnp.concatenate(idxs, axis=0)


_GAUSS_T = np.ascontiguousarray(
    np.asarray(
        jax.device_get(
            jax.random.normal(
                jax.random.key(42), (TOKENS, NUM_EXPERTS), dtype=jnp.float32
            )
        )
    ).T
)


@jax.jit
def _router(x, w_comb, b_comb):
    gauss_t = jnp.asarray(_GAUSS_T)
    n_blocks = TOKENS // _BLOCK_T
    out_shape = (
        jax.ShapeDtypeStruct((NUM_EXPERTS, TOKENS), jnp.float32),
        jax.ShapeDtypeStruct((TOP_K, TOKENS), jnp.int32),
    )
    return pl.pallas_call(
        _router_block,
        grid=(n_blocks,),
        in_specs=[
            pl.BlockSpec((_BLOCK_T, N_EMBED), lambda i: (i, 0)),
            pl.BlockSpec((N_EMBED, 2 * NUM_EXPERTS), lambda i: (0, 0)),
            pl.BlockSpec((2 * NUM_EXPERTS, 1), lambda i: (0, 0)),
            pl.BlockSpec((NUM_EXPERTS, _BLOCK_T), lambda i: (0, i)),
        ],
        out_specs=(
            pl.BlockSpec((NUM_EXPERTS, _BLOCK_T), lambda i: (0, i)),
            pl.BlockSpec((TOP_K, _BLOCK_T), lambda i: (0, i)),
        ),
        out_shape=out_shape,
    )(x, w_comb, b_comb, gauss_t)


def kernel(x, W_router, b_router, W_noise, b_noise):
    w_comb = jnp.concatenate([W_router.T, W_noise.T], axis=1)
    b_comb = jnp.concatenate([b_router, b_noise])[:, None]
    out_t, idx_t = _router(x, w_comb, b_comb)
    return (out_t.T, idx_t.T)
